# Initial kernel scaffold; baseline (speedup 1.0000x reference)
import jax
import jax.numpy as jnp
from jax.experimental import pallas as pl


def kernel(dense_features, sparse_indices, tables, dW1, db1, dW2, db2, dW3, db3, oW1, ob1, oW2, ob2, oW3, ob3):
    raise NotImplementedError("write your pallas kernel here")


# speedup vs baseline: 2.2526x; 2.2526x over previous
import functools

import jax
import jax.numpy as jnp
from jax import lax
from jax.experimental import pallas as pl
from jax.experimental.pallas import tpu as pltpu
from jax.experimental.pallas import tpu_sc as plsc

_B = 4096
_F = 26
_V = 100000
_D = 32

_NC = 2
_NS = 16
_NW = _NC * _NS
_ROWS = _B * _F
_RPW = _ROWS // _NW
_CHUNK = 128
_NCHUNK = _RPW // _CHUNK


def _sc_gather_body(tbl, idx_hbm, off_hbm, out_hbm, idx_v, off_v, rows_v, sem):
    wid = lax.axis_index("s") * _NC + lax.axis_index("c")
    base = wid * _RPW
    pltpu.sync_copy(idx_hbm.at[pl.ds(base, _RPW)], idx_v)
    pltpu.sync_copy(off_hbm, off_v)

    def _add(i, carry):
        for u in range(4):
            s = pl.ds((i * 4 + u) * 16, 16)
            idx_v[s] = idx_v[s] + off_v[s]
        return carry

    lax.fori_loop(0, _RPW // 64, _add, 0)

    def _fire(c, carry):
        pltpu.make_async_copy(
            tbl.at[idx_v.at[pl.ds(c * _CHUNK, _CHUNK)]],
            rows_v.at[pl.ds(c * _CHUNK, _CHUNK)],
            sem,
        ).start()
        return carry

    lax.fori_loop(0, _NCHUNK, _fire, 0)

    def _drain(c, carry):
        pltpu.make_async_copy(
            tbl.at[idx_v.at[pl.ds(c * _CHUNK, _CHUNK)]],
            rows_v.at[pl.ds(c * _CHUNK, _CHUNK)],
            sem,
        ).wait()
        return carry

    lax.fori_loop(0, _NCHUNK, _drain, 0)

    pltpu.sync_copy(rows_v, out_hbm.at[pl.ds(base, _RPW)])


@functools.cache
def _sc_gather():
    mesh = plsc.VectorSubcoreMesh(core_axis_name="c", subcore_axis_name="s",
                                  num_cores=_NC, num_subcores=_NS)
    return pl.kernel(
        _sc_gather_body,
        out_type=jax.ShapeDtypeStruct((_ROWS, _D), jnp.float32),
        mesh=mesh,
        scratch_types=[
            pltpu.VMEM((_RPW,), jnp.int32),
            pltpu.VMEM((_RPW,), jnp.int32),
            pltpu.VMEM((_RPW, _D), jnp.float32),
            pltpu.SemaphoreType.DMA,
        ],
        compiler_params=pltpu.CompilerParams(use_tc_tiling_on_sc=False),
    )


_BB = 512
_NBLK = _B // _BB
_NFEAT = _F + 1


def _tc_body(xT, embT, w1t, b1, w2t, b2, w3t, b3,
             ow1t, ob1, ow2t, ob2, ow3t, ob3, out_ref):
    f32 = jnp.float32
    d = jnp.maximum(jnp.dot(w1t[...], xT[...], preferred_element_type=f32) + b1[...], 0.0)
    d = jnp.maximum(jnp.dot(w2t[...], d, preferred_element_type=f32) + b2[...], 0.0)
    d = jnp.maximum(jnp.dot(w3t[...], d, preferred_element_type=f32) + b3[...], 0.0)

    ct = jnp.concatenate([d, embT[...]], axis=0)

    pieces = [d]
    for f in range(_NFEAT - 1):
        g = _NFEAT - 1 - f
        e = ct[32 * f:32 * (f + 1), :]
        rest = ct[32 * (f + 1):, :].reshape(g, 32, _BB)
        pieces.append(jnp.sum(rest * e[None], axis=1))
    x = jnp.concatenate(pieces, axis=0)

    o = jnp.maximum(jnp.dot(ow1t[...], x, preferred_element_type=f32) + ob1[...], 0.0)
    o = jnp.maximum(jnp.dot(ow2t[...], o, preferred_element_type=f32) + ob2[...], 0.0)
    out_ref[...] = jnp.dot(ow3t[...], o, preferred_element_type=f32) + ob3[...]


def _full(shape):
    return pl.BlockSpec(shape, lambda j: (0, 0))


_tc_forward = pl.pallas_call(
    _tc_body,
    grid=(_NBLK,),
    in_specs=[
        pl.BlockSpec((13, _BB), lambda j: (0, j)),
        pl.BlockSpec((_F * _D, _BB), lambda j: (0, j)),
        _full((512, 13)), _full((512, 1)),
        _full((256, 512)), _full((256, 1)),
        _full((32, 256)), _full((32, 1)),
        _full((512, 383)), _full((512, 1)),
        _full((256, 512)), _full((256, 1)),
        _full((1, 256)), _full((1, 1)),
    ],
    out_specs=pl.BlockSpec((1, _BB), lambda j: (0, j)),
    out_shape=jax.ShapeDtypeStruct((1, _B), jnp.float32),
)


def kernel(dense_features, sparse_indices, tables, dW1, db1, dW2, db2, dW3, db3,
           oW1, ob1, oW2, ob2, oW3, ob3):
    tbl_flat = tables.reshape(_F * _V, _D)
    idx_flat = sparse_indices.reshape(-1)
    offs = jnp.tile(jnp.arange(_F, dtype=jnp.int32) * _V, _RPW // _F)

    emb = _sc_gather()(tbl_flat, idx_flat, offs)
    embT = emb.reshape(_B, _F * _D).T

    out_t = _tc_forward(
        dense_features.T, embT,
        dW1.T, db1.reshape(-1, 1), dW2.T, db2.reshape(-1, 1),
        dW3.T, db3.reshape(-1, 1),
        oW1.T, ob1.reshape(-1, 1), oW2.T, ob2.reshape(-1, 1),
        oW3.T, ob3.reshape(-1, 1),
    )
    return out_t.reshape(_B, 1)


# speedup vs baseline: 2.2958x; 1.0191x over previous
import functools

import jax
import jax.numpy as jnp
from jax import lax
from jax.experimental import pallas as pl
from jax.experimental.pallas import tpu as pltpu
from jax.experimental.pallas import tpu_sc as plsc

_B = 4096
_F = 26
_V = 100000
_D = 32

_NC = 2
_NS = 16
_NW = _NC * _NS
_ROWS = _B * _F
_RPW = _ROWS // _NW
_CHUNK = 128
_NCHUNK = _RPW // _CHUNK


def _sc_gather_body(tbl, idx_hbm, off_hbm, out_hbm, idx_v, off_v, rows_v, sem):
    wid = lax.axis_index("s") * _NC + lax.axis_index("c")
    base = wid * _RPW
    pltpu.sync_copy(idx_hbm.at[pl.ds(base, _RPW)], idx_v)
    pltpu.sync_copy(off_hbm, off_v)

    def _add(i, carry):
        for u in range(4):
            s = pl.ds((i * 4 + u) * 16, 16)
            idx_v[s] = idx_v[s] + off_v[s]
        return carry

    lax.fori_loop(0, _RPW // 64, _add, 0)

    def _fire(c, carry):
        pltpu.make_async_copy(
            tbl.at[idx_v.at[pl.ds(c * _CHUNK, _CHUNK)]],
            rows_v.at[pl.ds(c * _CHUNK, _CHUNK)],
            sem,
        ).start()
        return carry

    lax.fori_loop(0, _NCHUNK, _fire, 0)

    def _drain(c, carry):
        pltpu.make_async_copy(
            tbl.at[idx_v.at[pl.ds(c * _CHUNK, _CHUNK)]],
            rows_v.at[pl.ds(c * _CHUNK, _CHUNK)],
            sem,
        ).wait()
        return carry

    lax.fori_loop(0, _NCHUNK, _drain, 0)

    pltpu.sync_copy(rows_v, out_hbm.at[pl.ds(base, _RPW)])


@functools.cache
def _sc_gather():
    mesh = plsc.VectorSubcoreMesh(core_axis_name="c", subcore_axis_name="s",
                                  num_cores=_NC, num_subcores=_NS)
    return pl.kernel(
        _sc_gather_body,
        out_type=jax.ShapeDtypeStruct((_ROWS, _D), jnp.float32),
        mesh=mesh,
        scratch_types=[
            pltpu.VMEM((_RPW,), jnp.int32),
            pltpu.VMEM((_RPW,), jnp.int32),
            pltpu.VMEM((_RPW, _D), jnp.float32),
            pltpu.SemaphoreType.DMA,
        ],
        compiler_params=pltpu.CompilerParams(use_tc_tiling_on_sc=False),
    )


_BB = 512
_NBLK = _B // _BB
_NFEAT = _F + 1


def _tc_body(xT, embT, w1t, b1, w2t, b2, w3t, b3,
             ow1t, ob1, ow2t, ob2, ow3t, ob3, out_ref):
    f32 = jnp.float32
    d = jnp.maximum(jnp.dot(w1t[...], xT[...], preferred_element_type=f32) + b1[...], 0.0)
    d = jnp.maximum(jnp.dot(w2t[...], d, preferred_element_type=f32) + b2[...], 0.0)
    d = jnp.maximum(jnp.dot(w3t[...], d, preferred_element_type=f32) + b3[...], 0.0)

    ct = jnp.concatenate([d, jnp.transpose(embT[...])], axis=0)

    pieces = [d]
    for f in range(_NFEAT - 1):
        g = _NFEAT - 1 - f
        e = ct[32 * f:32 * (f + 1), :]
        rest = ct[32 * (f + 1):, :].reshape(g, 32, _BB)
        pieces.append(jnp.sum(rest * e[None], axis=1))
    x = jnp.concatenate(pieces, axis=0)

    o = jnp.maximum(jnp.dot(ow1t[...], x, preferred_element_type=f32) + ob1[...], 0.0)
    o = jnp.maximum(jnp.dot(ow2t[...], o, preferred_element_type=f32) + ob2[...], 0.0)
    out_ref[...] = jnp.dot(ow3t[...], o, preferred_element_type=f32) + ob3[...]


def _full(shape):
    return pl.BlockSpec(shape, lambda j: (0, 0))


_tc_forward = pl.pallas_call(
    _tc_body,
    grid=(_NBLK,),
    in_specs=[
        pl.BlockSpec((13, _BB), lambda j: (0, j)),
        pl.BlockSpec((_BB, _F * _D), lambda j: (j, 0)),
        _full((512, 13)), _full((512, 1)),
        _full((256, 512)), _full((256, 1)),
        _full((32, 256)), _full((32, 1)),
        _full((512, 383)), _full((512, 1)),
        _full((256, 512)), _full((256, 1)),
        _full((1, 256)), _full((1, 1)),
    ],
    out_specs=pl.BlockSpec((1, _BB), lambda j: (0, j)),
    out_shape=jax.ShapeDtypeStruct((1, _B), jnp.float32),
)


def kernel(dense_features, sparse_indices, tables, dW1, db1, dW2, db2, dW3, db3,
           oW1, ob1, oW2, ob2, oW3, ob3):
    tbl_flat = tables.reshape(_F * _V, _D)
    idx_flat = sparse_indices.reshape(-1)
    offs = jnp.tile(jnp.arange(_F, dtype=jnp.int32) * _V, _RPW // _F)

    emb = _sc_gather()(tbl_flat, idx_flat, offs)
    emb2 = emb.reshape(_B, _F * _D)

    out_t = _tc_forward(
        dense_features.T, emb2,
        dW1.T, db1.reshape(-1, 1), dW2.T, db2.reshape(-1, 1),
        dW3.T, db3.reshape(-1, 1),
        oW1.T, ob1.reshape(-1, 1), oW2.T, ob2.reshape(-1, 1),
        oW3.T, ob3.reshape(-1, 1),
    )
    return out_t.reshape(_B, 1)
